# Initial kernel scaffold; baseline (speedup 1.0000x reference)
import jax
import jax.numpy as jnp
from jax.experimental import pallas as pl


def kernel(grid, node_w, node_b, edge_w, edge_b, msg_w1, msg_b1, msg_w2, msg_b2, upd_w1, upd_b1, upd_w2, upd_b2, out_w, out_b):
    raise NotImplementedError("write your pallas kernel here")


# speedup vs baseline: 166.5353x; 166.5353x over previous
import functools

import jax
import jax.numpy as jnp
from jax import lax
from jax.experimental import pallas as pl
from jax.experimental.pallas import tpu as pltpu

_D = 4
_L = 6
_H = 128
_W = 128
_NCLS = 10
_DIRS = ((-1, 0), (1, 0), (0, -1), (0, 1))


def _shift(a, di, dj):
    f32 = jnp.float32
    if di == -1:
        a = jnp.concatenate([jnp.zeros((1, _W), f32), a[:-1, :]], axis=0)
    elif di == 1:
        a = jnp.concatenate([a[1:, :], jnp.zeros((1, _W), f32)], axis=0)
    if dj == -1:
        a = jnp.concatenate([jnp.zeros((_H, 1), f32), a[:, :-1]], axis=1)
    elif dj == 1:
        a = jnp.concatenate([a[:, 1:], jnp.zeros((_H, 1), f32)], axis=1)
    return a


def _gnn_kernel(grid_ref, node_w, node_b, edge_w, edge_b, mw1, mb1, mw2, mb2,
                uw1, ub1, uw2, ub2, ow, ob, out_ref):
    f32 = jnp.float32
    g = grid_ref[...].astype(f32)

    x = [g * node_w[c, 0] + node_b[c] for c in range(_D)]

    ii = lax.broadcasted_iota(jnp.int32, (_H, _W), 0).astype(f32) * (1.0 / _H)
    jj = lax.broadcasted_iota(jnp.int32, (_H, _W), 1).astype(f32) * (1.0 / _W)
    epos = [ii * edge_w[c, 2] + jj * edge_w[c, 3] + edge_b[c] for c in range(_D)]

    for l in range(_L):
        p = []
        for h in range(_D):
            acc = x[0] * mw1[l, h, 0]
            for c in range(1, _D):
                acc = acc + x[c] * mw1[l, h, c]
            for c in range(_D):
                acc = acc + epos[c] * mw1[l, h, 2 * _D + c]
            p.append(acc + mb1[l, h])

        agg = [None] * _D
        for (di, dj) in _DIRS:
            tf = [_shift(x[c], di, dj) for c in range(_D)]
            hid = []
            for h in range(_D):
                alpha = mw1[l, h, 2 * _D] * (di * edge_w[0, 0] + dj * edge_w[0, 1])
                for c in range(1, _D):
                    alpha = alpha + mw1[l, h, 2 * _D + c] * (
                        di * edge_w[c, 0] + dj * edge_w[c, 1])
                acc = p[h] + alpha
                for c in range(_D):
                    acc = acc + tf[c] * mw1[l, h, _D + c]
                hid.append(jnp.maximum(acc, 0.0))
            for c in range(_D):
                m = hid[0] * mw2[l, c, 0]
                for h in range(1, _D):
                    m = m + hid[h] * mw2[l, c, h]
                m = m + mb2[l, c]
                sm = _shift(m, -di, -dj)
                agg[c] = sm if agg[c] is None else agg[c] + sm

        hid2 = []
        for h in range(_D):
            acc = x[0] * uw1[l, h, 0]
            for c in range(1, _D):
                acc = acc + x[c] * uw1[l, h, c]
            for c in range(_D):
                acc = acc + agg[c] * uw1[l, h, _D + c]
            hid2.append(jnp.maximum(acc + ub1[l, h], 0.0))
        newx = []
        for c in range(_D):
            acc = hid2[0] * uw2[l, c, 0]
            for h in range(1, _D):
                acc = acc + hid2[h] * uw2[l, c, h]
            newx.append(acc + ub2[l, c])
        x = newx

    for k in range(_NCLS):
        acc = x[0] * ow[k, 0]
        for c in range(1, _D):
            acc = acc + x[c] * ow[k, c]
        out_ref[k, :, :] = acc + ob[k]


@functools.partial(jax.jit, static_argnames=())
def kernel(grid, node_w, node_b, edge_w, edge_b, msg_w1, msg_b1, msg_w2,
           msg_b2, upd_w1, upd_b1, upd_w2, upd_b2, out_w, out_b):
    smem = pl.BlockSpec(memory_space=pltpu.SMEM)
    vmem = pl.BlockSpec(memory_space=pltpu.VMEM)
    out = pl.pallas_call(
        _gnn_kernel,
        out_shape=jax.ShapeDtypeStruct((_NCLS, _H, _W), jnp.float32),
        in_specs=[vmem] + [smem] * 14,
        out_specs=vmem,
    )(grid, node_w, node_b, edge_w, edge_b, msg_w1, msg_b1, msg_w2, msg_b2,
      upd_w1, upd_b1, upd_w2, upd_b2, out_w, out_b)
    return jnp.transpose(out, (1, 2, 0))
